# Initial kernel scaffold; baseline (speedup 1.0000x reference)
import jax
import jax.numpy as jnp
from jax.experimental import pallas as pl


def kernel(pos_input, pos_item, neg_item, input_embeddings, item_embeddings):
    raise NotImplementedError("write your pallas kernel here")


# speedup vs baseline: 4.6837x; 4.6837x over previous
import functools

import jax
import jax.numpy as jnp
from jax import lax
from jax.experimental import pallas as pl
from jax.experimental.pallas import tpu as pltpu
from jax.experimental.pallas import tpu_sc as plsc

B = 16384
D = 64
NEG = 20
COLS = NEG + 1
NC, NS = 2, 16
NW = NC * NS
RPW = B // NW
C = 64
NCHUNK = RPW // C
NEG_ROWS = C * NEG
NIDX_W = 128
NSTREAM = NEG_ROWS // NIDX_W
NG_ROWS_PER_W = RPW * NEG // NIDX_W


def _sc_scores(pos_input, pos_item, neg_idx2d, input_embeddings,
               item_embeddings):
    mesh = plsc.VectorSubcoreMesh(core_axis_name="c", subcore_axis_name="s")

    @functools.partial(
        pl.kernel,
        out_type=jax.ShapeDtypeStruct((B * COLS,), jnp.float32),
        mesh=mesh,
        scratch_types=[
            pltpu.VMEM((C,), jnp.int32),
            pltpu.VMEM((C,), jnp.int32),
            pltpu.VMEM((NEG_ROWS,), jnp.int32),
            pltpu.VMEM((C, D), jnp.float32),
            pltpu.VMEM((C, D), jnp.float32),
            pltpu.VMEM((NEG_ROWS, D), jnp.float32),
            pltpu.VMEM((C * COLS,), jnp.float32),
            pltpu.SemaphoreType.DMA,
        ],
        compiler_params=pltpu.CompilerParams(needs_layout_passes=False,
                                             use_tc_tiling_on_sc=False),
    )
    def sc_kernel(pi_hbm, it_hbm, ng_hbm, utab_hbm, itab_hbm, out_hbm,
                  pi_v, it_v, ng_v, user_v, item_v, neg_v, sc_v, sem):
        wid = lax.axis_index("s") * NC + lax.axis_index("c")
        lane0 = lax.iota(jnp.int32, 16) < 1

        def chunk_body(ci, carry):
            base = wid * RPW + ci * C
            pltpu.sync_copy(pi_hbm.at[pl.ds(base, C)], pi_v)
            pltpu.sync_copy(it_hbm.at[pl.ds(base, C)], it_v)
            pltpu.sync_copy(ng_hbm.at[pl.ds(base * NEG, NEG_ROWS)], ng_v)
            copies = [
                pltpu.async_copy(utab_hbm.at[pi_v], user_v, sem),
                pltpu.async_copy(itab_hbm.at[it_v], item_v, sem),
            ]
            for j in range(NSTREAM):
                copies.append(
                    pltpu.async_copy(itab_hbm.at[ng_v.at[pl.ds(j * NIDX_W,
                                                               NIDX_W)]],
                                     neg_v.at[pl.ds(j * NIDX_W, NIDX_W)],
                                     sem))
            for cpy in copies:
                cpy.wait()

            def row_body(r, rcarry):
                u0 = user_v[r, pl.ds(0, 16)]
                u1 = user_v[r, pl.ds(16, 16)]
                u2 = user_v[r, pl.ds(32, 16)]
                u3 = user_v[r, pl.ds(48, 16)]
                acc = (item_v[r, pl.ds(0, 16)] * u0
                       + item_v[r, pl.ds(16, 16)] * u1
                       + item_v[r, pl.ds(32, 16)] * u2
                       + item_v[r, pl.ds(48, 16)] * u3)
                s0 = -jnp.sum(acc)
                plsc.store_scatter(
                    sc_v, [jnp.full((16,), r * COLS, jnp.int32)],
                    jnp.full((16,), s0, jnp.float32), mask=lane0)

                def neg_body(j, ncarry):
                    nr = r * NEG + j
                    a = (neg_v[nr, pl.ds(0, 16)] * u0
                         + neg_v[nr, pl.ds(16, 16)] * u1
                         + neg_v[nr, pl.ds(32, 16)] * u2
                         + neg_v[nr, pl.ds(48, 16)] * u3)
                    s = jnp.sum(a)
                    plsc.store_scatter(
                        sc_v, [jnp.full((16,), r * COLS + 1 + j, jnp.int32)],
                        jnp.full((16,), s, jnp.float32), mask=lane0)
                    return ncarry

                lax.fori_loop(0, NEG, neg_body, 0, unroll=5)
                return rcarry

            lax.fori_loop(0, C, row_body, 0)
            pltpu.sync_copy(sc_v, out_hbm.at[pl.ds(base * COLS, C * COLS)])
            return carry

        lax.fori_loop(0, NCHUNK, chunk_body, 0)

    return sc_kernel(pos_input, pos_item, neg_idx2d, input_embeddings,
                     item_embeddings)


def _tc_reduce(scores):
    def body(s_ref, o_ref):
        x = s_ref[...]
        xc = jnp.clip(x, -10.0, 10.0)
        sp = jnp.log(1.0 + jnp.exp(xc))
        o_ref[0, 0] = jnp.sum(sp) * (1.0 / B)

    return pl.pallas_call(
        body,
        out_shape=jax.ShapeDtypeStruct((1, 1), jnp.float32),
        out_specs=pl.BlockSpec(memory_space=pltpu.SMEM),
    )(scores)


def kernel(pos_input, pos_item, neg_item, input_embeddings, item_embeddings):
    pi = pos_input.astype(jnp.int32)
    it = pos_item.astype(jnp.int32)
    ng = neg_item.astype(jnp.int32).reshape(B * NEG)
    scores = _sc_scores(pi, it, ng, input_embeddings, item_embeddings)
    return _tc_reduce(scores.reshape(B, COLS))[0, 0]
